# baseline (speedup 1.0000x reference)
import jax
import jax.numpy as jnp
from jax.experimental import pallas as pl
from jax.experimental.pallas import tpu as pltpu

LEAK = 0.6
BIAS = 1.6

_N_TILE = 256


def _esn_body(u_ref, s_ref, wr_ref, o_ref, s_bf16_ref):
    j = pl.program_id(0)

    @pl.when(j == 0)
    def _():
        s_bf16_ref[...] = s_ref[...].astype(jnp.bfloat16)

    pre = jax.lax.dot_general(
        s_bf16_ref[...],
        wr_ref[...].astype(jnp.bfloat16),
        dimension_numbers=(((1,), (1,)), ((), ())),
        preferred_element_type=jnp.float32,
    )
    pre = pre + u_ref[...] + BIAS
    s_tile = s_ref[:, pl.ds(j * _N_TILE, _N_TILE)]
    o_ref[...] = LEAK * jnp.tanh(pre) + (1.0 - LEAK) * s_tile


@jax.jit
def kernel(proj_vars, res_state, wr):
    batch, res_dim = res_state.shape
    n_tiles = wr.shape[0] // _N_TILE
    return pl.pallas_call(
        _esn_body,
        grid=(n_tiles,),
        in_specs=[
            pl.BlockSpec((batch, _N_TILE), lambda j: (0, j)),
            pl.BlockSpec((batch, res_dim), lambda j: (0, 0)),
            pl.BlockSpec((_N_TILE, res_dim), lambda j: (j, 0)),
        ],
        out_specs=pl.BlockSpec((batch, _N_TILE), lambda j: (0, j)),
        out_shape=jax.ShapeDtypeStruct((batch, wr.shape[0]), jnp.float32),
        scratch_shapes=[pltpu.VMEM((batch, res_dim), jnp.bfloat16)],
    )(proj_vars, res_state, wr)


# speedup vs baseline: 1.0503x; 1.0503x over previous
import jax
import jax.numpy as jnp
from jax.experimental import pallas as pl

LEAK = 0.6
BIAS = 1.6

_N_TILE = 512


def _esn_body(u_ref, s_ref, wr_ref, o_ref):
    j = pl.program_id(0)
    pre = jax.lax.dot_general(
        s_ref[...],
        wr_ref[...],
        dimension_numbers=(((1,), (1,)), ((), ())),
        preferred_element_type=jnp.float32,
        precision=jax.lax.Precision.DEFAULT,
    )
    pre = pre + u_ref[...] + BIAS
    s_tile = s_ref[:, pl.ds(j * _N_TILE, _N_TILE)]
    o_ref[...] = LEAK * jnp.tanh(pre) + (1.0 - LEAK) * s_tile


@jax.jit
def kernel(proj_vars, res_state, wr):
    batch, res_dim = res_state.shape
    n_tiles = wr.shape[0] // _N_TILE
    return pl.pallas_call(
        _esn_body,
        grid=(n_tiles,),
        in_specs=[
            pl.BlockSpec((batch, _N_TILE), lambda j: (0, j)),
            pl.BlockSpec((batch, res_dim), lambda j: (0, 0)),
            pl.BlockSpec((_N_TILE, res_dim), lambda j: (j, 0)),
        ],
        out_specs=pl.BlockSpec((batch, _N_TILE), lambda j: (0, j)),
        out_shape=jax.ShapeDtypeStruct((batch, wr.shape[0]), jnp.float32),
    )(proj_vars, res_state, wr)
